# Initial kernel scaffold; baseline (speedup 1.0000x reference)
import jax
import jax.numpy as jnp
from jax.experimental import pallas as pl


def kernel(x):
    raise NotImplementedError("write your pallas kernel here")


# speedup vs baseline: 1.6069x; 1.6069x over previous
import functools

import jax
import jax.numpy as jnp
from jax import lax
from jax.experimental import pallas as pl
from jax.experimental.pallas import tpu as pltpu
from jax.experimental.pallas import tpu_sc as plsc

_B, _N, _D = 16, 4096, 1280
_BN = 128


def _maxpool_body(x_ref, pooled_ref, idx_ref):
    j = pl.program_id(0)
    xb = x_ref[...]
    bm = jnp.max(xb, axis=1)
    iota = lax.broadcasted_iota(jnp.int32, xb.shape, 1)
    bi = jnp.min(jnp.where(xb == bm[:, None, :], iota, _BN), axis=1) + j * _BN

    @pl.when(j == 0)
    def _init():
        pooled_ref[...] = bm
        idx_ref[...] = bi

    @pl.when(j > 0)
    def _merge():
        m = pooled_ref[...]
        take = bm > m
        pooled_ref[...] = jnp.where(take, bm, m)
        idx_ref[...] = jnp.where(take, bi, idx_ref[...])


def _tc_maxpool(x):
    nj = _N // _BN
    return pl.pallas_call(
        _maxpool_body,
        grid=(nj,),
        in_specs=[pl.BlockSpec((_B, _BN, _D), lambda j: (0, j, 0))],
        out_specs=[
            pl.BlockSpec((_B, _D), lambda j: (0, 0)),
            pl.BlockSpec((_B, _D), lambda j: (0, 0)),
        ],
        out_shape=[
            jax.ShapeDtypeStruct((_B, _D), jnp.float32),
            jax.ShapeDtypeStruct((_B, _D), jnp.int32),
        ],
    )(x)


def _sc_scatter_body(idx_hbm, out_hbm, idx_v, row_v):
    wid = lax.axis_index("s") * 2 + lax.axis_index("c")

    @pl.when(wid < _B)
    def _():
        pltpu.sync_copy(idx_hbm.at[wid], idx_v)
        zeros16 = jnp.zeros((16,), jnp.float32)

        def zbody(i, carry):
            row_v[pl.ds(i * 16, 16)] = zeros16
            return carry

        lax.fori_loop(0, _N // 16, zbody, 0)
        ones16 = jnp.ones((16,), jnp.float32)

        def sbody(i, carry):
            iv = idx_v[pl.ds(i * 16, 16)]
            plsc.store_scatter(row_v, [iv], ones16)
            return carry

        lax.fori_loop(0, _D // 16, sbody, 0)
        pltpu.sync_copy(row_v, out_hbm.at[wid])


def _sc_scatter(idx):
    call = pl.kernel(
        _sc_scatter_body,
        mesh=plsc.VectorSubcoreMesh(core_axis_name="c", subcore_axis_name="s"),
        compiler_params=pltpu.CompilerParams(needs_layout_passes=False),
        out_type=jax.ShapeDtypeStruct((_B, _N), jnp.float32),
        scratch_types=[
            pltpu.VMEM((_D,), jnp.int32),
            pltpu.VMEM((_N,), jnp.float32),
        ],
    )
    return call(idx)


def kernel(x):
    pooled, idx = _tc_maxpool(x)
    attn = _sc_scatter(idx)
    return pooled, attn
